# baseline (speedup 1.0000x reference)
import jax
import jax.numpy as jnp
from jax.experimental import pallas as pl
from jax.experimental.pallas import tpu as pltpu


def _gather_body(lens_ref, in_hbm, out_hbm, rows_vmem, sem):
    B, T, _ = in_hbm.shape
    copies = []
    for b in range(B):
        n = lens_ref[b]
        row = jnp.where(n > 0, n - 1, T - 1)
        cp = pltpu.make_async_copy(in_hbm.at[b, row], rows_vmem.at[b, 0], sem)
        cp.start()
        copies.append(cp)
    for cp in copies:
        cp.wait()
    pltpu.make_async_copy(rows_vmem, out_hbm, sem).start()
    pltpu.make_async_copy(rows_vmem, out_hbm, sem).wait()


def kernel(input_, lengths):
    B, T, D = input_.shape
    lens = lengths.astype(jnp.int32)

    return pl.pallas_call(
        _gather_body,
        in_specs=[
            pl.BlockSpec(memory_space=pltpu.MemorySpace.SMEM),
            pl.BlockSpec(memory_space=pltpu.MemorySpace.HBM),
        ],
        out_specs=pl.BlockSpec(memory_space=pltpu.MemorySpace.HBM),
        scratch_shapes=[
            pltpu.VMEM((B, 1, D), jnp.float32),
            pltpu.SemaphoreType.DMA,
        ],
        out_shape=jax.ShapeDtypeStruct((B, 1, D), input_.dtype),
    )(lens, input_)
